# baseline (speedup 1.0000x reference)
import functools

import jax
import jax.numpy as jnp
from jax import lax
from jax.experimental import pallas as pl
from jax.experimental.pallas import tpu as pltpu
from jax.experimental.pallas import tpu_sc as plsc

_VOCAB = 100000
_B, _S = 32, 8
_ROWS = _B * _S
_R = 16
_NB = _ROWS // _R
_MARGIN = 1e-3
_SC_WORKERS = 32
_ROWS_PER_WORKER = _ROWS // _SC_WORKERS
_SEG = 16

_noise_cache = []


def _noise():
    if not _noise_cache:
        with jax.ensure_compile_time_eval():
            key = jax.random.key(42)
            k1, k2 = jax.random.split(key)
            shape = (_B, _S, _VOCAB)
            u = jax.random.uniform(k1, shape, dtype=jnp.float32) * (0.999 - 1e-12) + 1e-12
            g1 = -jnp.log(-jnp.log(u))
            g2 = jax.random.gumbel(k2, shape, dtype=jnp.float32)
            g1 = g1.reshape(_ROWS, _VOCAB)
            g2 = g2.reshape(_ROWS, _VOCAB)
            _noise_cache.append((g1, g2, g1 + g2))
    return _noise_cache[0]


def _fold_body(x_ref, gg_ref, ids_ref, gap_ref):
    z = x_ref[...] + gg_ref[...]
    m1 = jnp.max(z, axis=-1, keepdims=True)
    iota = jax.lax.broadcasted_iota(jnp.int32, z.shape, 1)
    idx = jnp.min(jnp.where(z == m1, iota, _VOCAB), axis=-1, keepdims=True)
    z2 = jnp.where(iota == idx, -jnp.inf, z)
    m2 = jnp.max(z2, axis=-1, keepdims=True)
    ids_ref[...] = idx[None]
    gap_ref[...] = (m1 - m2)[None]


def _exact_ids_body(x_ref, g1_ref, g2_ref, ids_ref):
    gl = x_ref[...] + g1_ref[...]
    m = jnp.max(gl, axis=-1, keepdims=True)
    e = jnp.exp(gl - m)
    s = jnp.sum(e, axis=-1, keepdims=True)
    y = e / s
    z = jnp.log(y + 1e-30) + g2_ref[...]
    zm = jnp.max(z, axis=-1, keepdims=True)
    iota = jax.lax.broadcasted_iota(jnp.int32, z.shape, 1)
    idx = jnp.min(jnp.where(z == zm, iota, _VOCAB), axis=-1, keepdims=True)
    ids_ref[...] = idx[None]


def _zero_fill():
    mesh = plsc.VectorSubcoreMesh(core_axis_name="c", subcore_axis_name="s")

    @functools.partial(
        pl.kernel,
        out_type=jax.ShapeDtypeStruct((_ROWS, _VOCAB), jnp.float32),
        mesh=mesh,
        scratch_types=[pltpu.VMEM((_VOCAB,), jnp.float32)],
    )
    def _zero(out_hbm, zbuf):
        wid = lax.axis_index("s") * 2 + lax.axis_index("c")

        def zbody(i, c):
            zbuf[pl.ds(i * 16, 16)] = jnp.zeros((16,), jnp.float32)
            return c

        lax.fori_loop(0, _VOCAB // 16, zbody, 0)

        def rbody(r, c):
            pltpu.sync_copy(zbuf, out_hbm.at[wid * _ROWS_PER_WORKER + r])
            return c

        lax.fori_loop(0, _ROWS_PER_WORKER, rbody, 0)

    return _zero()


def _patch_body(bidx_ref, loff_ref, oh_in_ref, oh_ref):
    del bidx_ref, oh_in_ref
    i = pl.program_id(0)
    sub = loff_ref[i]
    r_iota = jax.lax.broadcasted_iota(jnp.int32, (1, 8, 16), 1)
    c_iota = jax.lax.broadcasted_iota(jnp.int32, (1, 8, 16), 2)
    oh_ref[...] = jnp.where(r_iota * 16 + c_iota == sub, 1.0, 0.0)


def _row_spec():
    return pl.BlockSpec((_R, _VOCAB), lambda i: (i, 0))


def _ids_spec():
    return pl.BlockSpec((1, _R, 1), lambda i: (i, 0, 0))


def _ids_shape():
    return jax.ShapeDtypeStruct((_NB, _R, 1), jnp.int32)


def kernel(x):
    g1, g2, gg = _noise()
    x2 = x.reshape(_ROWS, _VOCAB)

    oh0 = _zero_fill()

    ids3, gaps = pl.pallas_call(
        _fold_body,
        grid=(_NB,),
        in_specs=[_row_spec(), _row_spec()],
        out_specs=[_ids_spec(), _ids_spec()],
        out_shape=[_ids_shape(),
                   jax.ShapeDtypeStruct((_NB, _R, 1), jnp.float32)],
    )(x2, gg)

    def _exact_ids():
        return pl.pallas_call(
            _exact_ids_body,
            grid=(_NB,),
            in_specs=[_row_spec(), _row_spec(), _row_spec()],
            out_specs=_ids_spec(),
            out_shape=_ids_shape(),
        )(x2, g1, g2)

    ids3 = jax.lax.cond(jnp.min(gaps) <= _MARGIN, _exact_ids, lambda: ids3)

    ids1 = ids3.reshape(_ROWS)
    bidx = ids1 // 128
    loff = ids1 % 128

    one_hot = pl.pallas_call(
        _patch_body,
        grid_spec=pltpu.PrefetchScalarGridSpec(
            num_scalar_prefetch=2,
            grid=(_ROWS,),
            in_specs=[pl.BlockSpec(memory_space=pl.ANY)],
            out_specs=pl.BlockSpec((1, 8, 16), lambda i, bref, lref: (i, bref[i], 0)),
        ),
        out_shape=jax.ShapeDtypeStruct((_ROWS, _VOCAB // 16, 16), jnp.float32),
        input_output_aliases={2: 0},
    )(bidx, loff, oh0.reshape(_ROWS, _VOCAB // 16, 16))

    message_ids = ids3.reshape(_B, _S)
    message_one_hot = one_hot.reshape(_B, _S, _VOCAB)
    eff_temperature = jnp.array([1.0], dtype=jnp.float32)
    return (message_ids, message_one_hot, eff_temperature)


# speedup vs baseline: 1.0488x; 1.0488x over previous
import functools

import jax
import jax.numpy as jnp
from jax import lax
from jax.experimental import pallas as pl
from jax.experimental.pallas import tpu as pltpu
from jax.experimental.pallas import tpu_sc as plsc

_VOCAB = 100000
_B, _S = 32, 8
_ROWS = _B * _S
_R = 16
_NB = _ROWS // _R
_MARGIN = 1e-3
_SC_WORKERS = 32
_ROWS_PER_WORKER = _ROWS // _SC_WORKERS
_SEG = 16

_noise_cache = []


def _noise():
    if not _noise_cache:
        with jax.ensure_compile_time_eval():
            key = jax.random.key(42)
            k1, k2 = jax.random.split(key)
            shape = (_B, _S, _VOCAB)
            u = jax.random.uniform(k1, shape, dtype=jnp.float32) * (0.999 - 1e-12) + 1e-12
            g1 = -jnp.log(-jnp.log(u))
            g2 = jax.random.gumbel(k2, shape, dtype=jnp.float32)
            g1 = g1.reshape(_ROWS, _VOCAB)
            g2 = g2.reshape(_ROWS, _VOCAB)
            _noise_cache.append((g1, g2, g1 + g2))
    return _noise_cache[0]


def _fold_body(x_ref, gg_ref, ids_ref, gap_ref):
    z = x_ref[...] + gg_ref[...]
    m1 = jnp.max(z, axis=-1, keepdims=True)
    iota = jax.lax.broadcasted_iota(jnp.int32, z.shape, 1)
    idx = jnp.min(jnp.where(z == m1, iota, _VOCAB), axis=-1, keepdims=True)
    z2 = jnp.where(iota == idx, -jnp.inf, z)
    m2 = jnp.max(z2, axis=-1, keepdims=True)
    ids_ref[...] = idx[None]
    gap_ref[...] = (m1 - m2)[None]


def _exact_ids_body(x_ref, g1_ref, g2_ref, ids_ref):
    gl = x_ref[...] + g1_ref[...]
    m = jnp.max(gl, axis=-1, keepdims=True)
    e = jnp.exp(gl - m)
    s = jnp.sum(e, axis=-1, keepdims=True)
    y = e / s
    z = jnp.log(y + 1e-30) + g2_ref[...]
    zm = jnp.max(z, axis=-1, keepdims=True)
    iota = jax.lax.broadcasted_iota(jnp.int32, z.shape, 1)
    idx = jnp.min(jnp.where(z == zm, iota, _VOCAB), axis=-1, keepdims=True)
    ids_ref[...] = idx[None]


def _zero_fill():
    mesh = plsc.VectorSubcoreMesh(core_axis_name="c", subcore_axis_name="s")

    @functools.partial(
        pl.kernel,
        out_type=jax.ShapeDtypeStruct((_ROWS, _VOCAB), jnp.float32),
        mesh=mesh,
        scratch_types=[pltpu.VMEM((_VOCAB,), jnp.float32)],
    )
    def _zero(out_hbm, zbuf):
        wid = lax.axis_index("s") * 2 + lax.axis_index("c")

        def zbody(i, c):
            zbuf[pl.ds(i * 16, 16)] = jnp.zeros((16,), jnp.float32)
            return c

        lax.fori_loop(0, _VOCAB // 16, zbody, 0)

        def rbody(r, c):
            pltpu.sync_copy(zbuf, out_hbm.at[wid * _ROWS_PER_WORKER + r])
            return c

        lax.fori_loop(0, _ROWS_PER_WORKER, rbody, 0)

    return _zero()


def _patch_body(ids_s_ref, ids_v_ref, oh_in_ref, oh_ref, seg_ref, sem):
    del oh_in_ref
    r_iota = jax.lax.broadcasted_iota(jnp.int32, (_ROWS, 8, 16), 1)
    c_iota = jax.lax.broadcasted_iota(jnp.int32, (_ROWS, 8, 16), 2)
    ids_v = ids_v_ref[...]
    seg_ref[...] = jnp.where(r_iota * 16 + c_iota == ids_v % 128, 1.0, 0.0)

    def _dma(r, tail):
        n = 2 if tail else 8
        idv = ids_s_ref[r, 0]
        g = (idv // 128) * 8
        return pltpu.make_async_copy(
            seg_ref.at[pl.ds(r, 1), pl.ds(0, n)],
            oh_ref.at[pl.ds(r, 1), pl.ds(g, n)],
            sem,
        )

    def sbody(r, c):
        tail = ids_s_ref[r, 0] >= (_VOCAB // 128) * 128

        @pl.when(tail)
        def _():
            _dma(r, True).start()

        @pl.when(jnp.logical_not(tail))
        def _():
            _dma(r, False).start()

        return c

    lax.fori_loop(0, _ROWS, sbody, 0)

    def wbody(r, c):
        tail = ids_s_ref[r, 0] >= (_VOCAB // 128) * 128

        @pl.when(tail)
        def _():
            _dma(r, True).wait()

        @pl.when(jnp.logical_not(tail))
        def _():
            _dma(r, False).wait()

        return c

    lax.fori_loop(0, _ROWS, wbody, 0)


def _row_spec():
    return pl.BlockSpec((_R, _VOCAB), lambda i: (i, 0))


def _ids_spec():
    return pl.BlockSpec((1, _R, 1), lambda i: (i, 0, 0))


def _ids_shape():
    return jax.ShapeDtypeStruct((_NB, _R, 1), jnp.int32)


def kernel(x):
    g1, g2, gg = _noise()
    x2 = x.reshape(_ROWS, _VOCAB)

    oh0 = _zero_fill()

    ids3, gaps = pl.pallas_call(
        _fold_body,
        grid=(_NB,),
        in_specs=[_row_spec(), _row_spec()],
        out_specs=[_ids_spec(), _ids_spec()],
        out_shape=[_ids_shape(),
                   jax.ShapeDtypeStruct((_NB, _R, 1), jnp.float32)],
    )(x2, gg)

    def _exact_ids():
        return pl.pallas_call(
            _exact_ids_body,
            grid=(_NB,),
            in_specs=[_row_spec(), _row_spec(), _row_spec()],
            out_specs=_ids_spec(),
            out_shape=_ids_shape(),
        )(x2, g1, g2)

    ids3 = jax.lax.cond(jnp.min(gaps) <= _MARGIN, _exact_ids, lambda: ids3)

    ids2 = ids3.reshape(_ROWS, 1)
    ids311 = ids3.reshape(_ROWS, 1, 1)

    one_hot = pl.pallas_call(
        _patch_body,
        in_specs=[
            pl.BlockSpec(memory_space=pltpu.SMEM),
            pl.BlockSpec(memory_space=pltpu.VMEM),
            pl.BlockSpec(memory_space=pl.ANY),
        ],
        out_specs=pl.BlockSpec(memory_space=pl.ANY),
        out_shape=jax.ShapeDtypeStruct((_ROWS, _VOCAB // 16, 16), jnp.float32),
        scratch_shapes=[
            pltpu.VMEM((_ROWS, 8, 16), jnp.float32),
            pltpu.SemaphoreType.DMA,
        ],
        input_output_aliases={2: 0},
    )(ids2, ids311, oh0.reshape(_ROWS, _VOCAB // 16, 16))

    message_ids = ids3.reshape(_B, _S)
    message_one_hot = one_hot.reshape(_B, _S, _VOCAB)
    eff_temperature = jnp.array([1.0], dtype=jnp.float32)
    return (message_ids, message_one_hot, eff_temperature)


# speedup vs baseline: 13.2265x; 12.6116x over previous
import jax
import jax.numpy as jnp
from jax.experimental import pallas as pl

_VOCAB = 100000
_B, _S = 32, 8
_ROWS = _B * _S
_R = 16
_NB = _ROWS // _R
_MARGIN = 1e-3

_noise_cache = []


def _noise():
    if not _noise_cache:
        with jax.ensure_compile_time_eval():
            key = jax.random.key(42)
            k1, k2 = jax.random.split(key)
            shape = (_B, _S, _VOCAB)
            u = jax.random.uniform(k1, shape, dtype=jnp.float32) * (0.999 - 1e-12) + 1e-12
            g1 = -jnp.log(-jnp.log(u))
            g2 = jax.random.gumbel(k2, shape, dtype=jnp.float32)
            g1 = g1.reshape(_ROWS, _VOCAB)
            g2 = g2.reshape(_ROWS, _VOCAB)
            _noise_cache.append((g1, g2, g1 + g2))
    return _noise_cache[0]


def _fold_body(x_ref, gg_ref, ids_ref, gap_ref):
    z = x_ref[...] + gg_ref[...]
    m1 = jnp.max(z, axis=-1, keepdims=True)
    iota = jax.lax.broadcasted_iota(jnp.int32, z.shape, 1)
    idx = jnp.min(jnp.where(z == m1, iota, _VOCAB), axis=-1, keepdims=True)
    z2 = jnp.where(iota == idx, -jnp.inf, z)
    m2 = jnp.max(z2, axis=-1, keepdims=True)
    ids_ref[...] = idx[None]
    gap_ref[...] = (m1 - m2)[None]


def _exact_ids_body(x_ref, g1_ref, g2_ref, ids_ref):
    gl = x_ref[...] + g1_ref[...]
    m = jnp.max(gl, axis=-1, keepdims=True)
    e = jnp.exp(gl - m)
    s = jnp.sum(e, axis=-1, keepdims=True)
    y = e / s
    z = jnp.log(y + 1e-30) + g2_ref[...]
    zm = jnp.max(z, axis=-1, keepdims=True)
    iota = jax.lax.broadcasted_iota(jnp.int32, z.shape, 1)
    idx = jnp.min(jnp.where(z == zm, iota, _VOCAB), axis=-1, keepdims=True)
    ids_ref[...] = idx[None]


def _onehot_body(ids_ref, oh_ref):
    idx = ids_ref[0]
    iota = jax.lax.broadcasted_iota(jnp.int32, (_R, _VOCAB), 1)
    oh_ref[...] = jnp.where(iota == idx, 1.0, 0.0)


def _row_spec():
    return pl.BlockSpec((_R, _VOCAB), lambda i: (i, 0))


def _ids_spec():
    return pl.BlockSpec((1, _R, 1), lambda i: (i, 0, 0))


def _ids_shape():
    return jax.ShapeDtypeStruct((_NB, _R, 1), jnp.int32)


def kernel(x):
    g1, g2, gg = _noise()
    x2 = x.reshape(_ROWS, _VOCAB)

    ids3, gaps = pl.pallas_call(
        _fold_body,
        grid=(_NB,),
        in_specs=[_row_spec(), _row_spec()],
        out_specs=[_ids_spec(), _ids_spec()],
        out_shape=[_ids_shape(),
                   jax.ShapeDtypeStruct((_NB, _R, 1), jnp.float32)],
    )(x2, gg)

    def _exact_ids():
        return pl.pallas_call(
            _exact_ids_body,
            grid=(_NB,),
            in_specs=[_row_spec(), _row_spec(), _row_spec()],
            out_specs=_ids_spec(),
            out_shape=_ids_shape(),
        )(x2, g1, g2)

    ids3 = jax.lax.cond(jnp.min(gaps) <= _MARGIN, _exact_ids, lambda: ids3)

    one_hot = pl.pallas_call(
        _onehot_body,
        grid=(_NB,),
        in_specs=[_ids_spec()],
        out_specs=_row_spec(),
        out_shape=jax.ShapeDtypeStruct((_ROWS, _VOCAB), jnp.float32),
    )(ids3)

    message_ids = ids3.reshape(_B, _S)
    message_one_hot = one_hot.reshape(_B, _S, _VOCAB)
    eff_temperature = jnp.array([1.0], dtype=jnp.float32)
    return (message_ids, message_one_hot, eff_temperature)
